# baseline (speedup 1.0000x reference)
import functools

import jax
import jax.numpy as jnp
from jax import lax
from jax.experimental import pallas as pl
from jax.experimental.pallas import tpu as pltpu
from jax.experimental.pallas import tpu_sc as plsc

N = 10000
E = 320000
IN_DIM = 128
HID = 64
ALIGN = 32
NUM_CLASSES = 7

NC = 2
NS = 16
NW = NC * NS

DEGPAD = 10240
DROWS = DEGPAD // NS
RPT = N // NS
CH = 128
NCHUNK = 2560
CPW = NCHUNK // NW
ACC_ROWS = DEGPAD
NB = 4
EBLK = 32768

BR = 2048
GRID = (N + BR - 1) // BR


def _sc_mesh():
    return plsc.VectorSubcoreMesh(core_axis_name="c", subcore_axis_name="s")


def _etr_body(e_ref, src_ref, dst_ref):
    i = pl.program_id(0)
    rows = EBLK // 128
    chunk2 = lax.broadcasted_iota(jnp.int32, (rows, 128), 0)
    lane = lax.broadcasted_iota(jnp.int32, (rows, 128), 1)
    elem = i * EBLK + chunk2 * 128 + lane
    valid = elem < E
    s2 = jnp.reshape(e_ref[0], (rows, 128))
    d2 = jnp.reshape(e_ref[1], (rows, 128))
    src_ref[...] = jnp.where(valid, s2, 0)
    dst_ref[...] = jnp.where(valid, d2, N + lane)


def _tc_etr(edge_index):
    rows = EBLK // 128
    return pl.pallas_call(
        _etr_body,
        grid=(NCHUNK // rows,),
        in_specs=[pl.BlockSpec((2, EBLK), lambda i: (0, i))],
        out_specs=[
            pl.BlockSpec((rows, 128), lambda i: (i, 0)),
            pl.BlockSpec((rows, 128), lambda i: (i, 0)),
        ],
        out_shape=[
            jax.ShapeDtypeStruct((NCHUNK, 128), jnp.int32),
            jax.ShapeDtypeStruct((NCHUNK, 128), jnp.int32),
        ],
    )(edge_index)


def _deg_body(dst_hbm, zeros_hbm, out0_hbm, out1_hbm, idx_v, ones_v, acc_sh, isem, asem):
    c = lax.axis_index("c")
    s = lax.axis_index("s")
    w = s * NC + c
    rslice = pl.ds(s * DROWS, DROWS)
    for i in range(8):
        ones_v[pl.ds(i * 16, 16)] = jnp.ones((16,), jnp.float32)
    pltpu.async_copy(zeros_hbm.at[rslice], acc_sh.at[rslice], isem).wait()
    pltpu.sync_copy(dst_hbm.at[pl.ds(w * CPW, CPW)], idx_v)
    plsc.subcore_barrier()

    @pl.loop(0, CPW, step=NB)
    def _chunks(t):
        hs = [
            pltpu.async_copy(ones_v, acc_sh.at[idx_v.at[t + b]],
                             asem, add=True)
            for b in range(NB)
        ]
        for h in hs:
            h.wait()

    plsc.subcore_barrier()

    @pl.when(c == 0)
    def _():
        pltpu.sync_copy(acc_sh.at[rslice], out0_hbm.at[rslice])

    @pl.when(c != 0)
    def _():
        pltpu.sync_copy(acc_sh.at[rslice], out1_hbm.at[rslice])


def _sc_degree(dst2d, zeros1d):
    return pl.kernel(
        _deg_body,
        out_type=[jax.ShapeDtypeStruct((DEGPAD,), jnp.float32),
                  jax.ShapeDtypeStruct((DEGPAD,), jnp.float32)],
        mesh=_sc_mesh(),
        compiler_params=pltpu.CompilerParams(use_tc_tiling_on_sc=False),
        scratch_types=[
            pltpu.VMEM((CPW, CH), jnp.int32),
            pltpu.VMEM((128,), jnp.float32),
            pltpu.VMEM_SHARED((DEGPAD,), jnp.float32),
            pltpu.SemaphoreType.DMA,
            pltpu.SemaphoreType.DMA,
        ],
    )(dst2d, zeros1d)


def _scat_body(y_hbm, src_hbm, dst_hbm, zeros_hbm, out_hbm,
               src_v, dst_v, rows_v, acc_sh, isem, gsem0, gsem1, ssem0, ssem1):
    c = lax.axis_index("c")
    s = lax.axis_index("s")
    w = s * NC + c
    rslice = pl.ds(s * RPT, RPT)
    gsems = (gsem0, gsem1)
    ssems = (ssem0, ssem1)
    NG = CPW // NB

    def fire_g(g, par):
        for b in range(NB):
            pltpu.async_copy(y_hbm.at[src_v.at[NB * g + b]],
                             rows_v.at[par * NB + b], gsems[par])

    def drain_g(g, par):
        for b in range(NB):
            pltpu.make_async_copy(y_hbm.at[src_v.at[NB * g + b]],
                                  rows_v.at[par * NB + b], gsems[par]).wait()

    def fire_s(g, par):
        for b in range(NB):
            pltpu.async_copy(rows_v.at[par * NB + b],
                             acc_sh.at[dst_v.at[NB * g + b]], ssems[par],
                             add=True)

    def drain_s(g, par):
        for b in range(NB):
            pltpu.make_async_copy(rows_v.at[par * NB + b],
                                  acc_sh.at[dst_v.at[NB * g + b]],
                                  ssems[par]).wait()

    arows = ACC_ROWS // NS
    ilo = s * arows
    tail_real = N - (NS - 1) * arows

    def _init_from(src_hbm):
        @pl.when(s < NS - 1)
        def _():
            pltpu.async_copy(src_hbm.at[pl.ds(ilo, arows)],
                             acc_sh.at[pl.ds(ilo, arows)], isem).wait()

        @pl.when(s == NS - 1)
        def _():
            h1 = pltpu.async_copy(src_hbm.at[pl.ds(ilo, tail_real)],
                                  acc_sh.at[pl.ds(ilo, tail_real)], isem)
            h2 = pltpu.async_copy(zeros_hbm.at[pl.ds(0, ACC_ROWS - N)],
                                  acc_sh.at[pl.ds(N, ACC_ROWS - N)], isem)
            h1.wait()
            h2.wait()

    @pl.when(c == 0)
    def _():
        _init_from(y_hbm)

    @pl.when(c != 0)
    def _():
        _init_from(zeros_hbm)

    pltpu.sync_copy(src_hbm.at[pl.ds(w * CPW, CPW)], src_v)
    pltpu.sync_copy(dst_hbm.at[pl.ds(w * CPW, CPW)], dst_v)
    plsc.subcore_barrier()

    fire_g(0, 0)
    fire_g(1, 1)
    drain_g(0, 0)
    fire_s(0, 0)

    @pl.loop(2, NG, step=2)
    def _groups(g):
        drain_s(g - 2, 0)
        fire_g(g, 0)
        drain_g(g - 1, 1)
        fire_s(g - 1, 1)
        drain_s(g - 1, 1)
        fire_g(g + 1, 1)
        drain_g(g, 0)
        fire_s(g, 0)

    drain_s(NG - 2, 0)
    drain_g(NG - 1, 1)
    fire_s(NG - 1, 1)
    drain_s(NG - 1, 1)

    plsc.subcore_barrier()
    pltpu.sync_copy(acc_sh.at[rslice],
                    out_hbm.at[pl.ds(s * RPT, RPT), pl.ds(c * HID, HID)])


def _sc_scatter(y, src2d, dst2d, zeros2d):
    return pl.kernel(
        _scat_body,
        out_type=jax.ShapeDtypeStruct((N, 2 * HID), jnp.float32),
        mesh=_sc_mesh(),
        compiler_params=pltpu.CompilerParams(use_tc_tiling_on_sc=False),
        scratch_types=[
            pltpu.VMEM((CPW, CH), jnp.int32),
            pltpu.VMEM((CPW, CH), jnp.int32),
            pltpu.VMEM((2 * NB, CH, HID), jnp.float32),
            pltpu.VMEM_SHARED((ACC_ROWS, HID), jnp.float32),
            pltpu.SemaphoreType.DMA,
            pltpu.SemaphoreType.DMA,
            pltpu.SemaphoreType.DMA,
            pltpu.SemaphoreType.DMA,
            pltpu.SemaphoreType.DMA,
        ],
    )(y, src2d, dst2d, zeros2d)


def _pre_body(x_ref, deg0_ref, deg1_ref, wg_ref, wps_ref, bps_ref,
              y_ref, zsem_ref):
    deg = deg0_ref[...] + deg1_ref[...] + 1.0
    dinv = lax.rsqrt(deg)
    xb = x_ref[...]
    xw = jnp.dot(xb, wg_ref[...], preferred_element_type=jnp.float32)
    y_ref[...] = xw * dinv[:, None]
    zs = (jnp.dot(xb, wps_ref[...], preferred_element_type=jnp.float32)
          + bps_ref[...])
    zsem_ref[...] = zs.T


def _tc_pre(xp, deg0, deg1, W_gcn, W_ps, b_ps2):
    return pl.pallas_call(
        _pre_body,
        grid=(GRID,),
        in_specs=[
            pl.BlockSpec((BR, IN_DIM), lambda i: (i, 0)),
            pl.BlockSpec((BR,), lambda i: (i,)),
            pl.BlockSpec((BR,), lambda i: (i,)),
            pl.BlockSpec((IN_DIM, HID), lambda i: (0, 0)),
            pl.BlockSpec((IN_DIM, ALIGN), lambda i: (0, 0)),
            pl.BlockSpec((1, ALIGN), lambda i: (0, 0)),
        ],
        out_specs=[
            pl.BlockSpec((BR, HID), lambda i: (i, 0)),
            pl.BlockSpec((ALIGN, BR), lambda i: (0, i)),
        ],
        out_shape=[
            jax.ShapeDtypeStruct((N, HID), jnp.float32),
            jax.ShapeDtypeStruct((ALIGN, N), jnp.float32),
        ],
    )(xp, deg0, deg1, W_gcn, W_ps, b_ps2)


def _post_body(acc_ref, deg0_ref, deg1_ref, zsem_ref, wpt_ref, wcls_ref,
               bg_ref, bpt_ref, bcls_ref, zt_ref, lg_ref, an_ref):
    deg = deg0_ref[...] + deg1_ref[...] + 1.0
    dinv = lax.rsqrt(deg)
    a2 = acc_ref[...]
    agg = (a2[:, :HID] + a2[:, HID:]) * dinv[:, None] + bg_ref[...]
    h = jnp.maximum(agg, 0.0)
    zt = jnp.dot(h, wpt_ref[...], preferred_element_type=jnp.float32) + bpt_ref[...]
    ztT = zt.T
    zt_ref[...] = ztT
    lg = (jnp.dot(zt, wcls_ref[...], preferred_element_type=jnp.float32)
          + bcls_ref[...])
    lg_ref[...] = lg.T
    diffT = ztT - zsem_ref[...]
    an_ref[...] = jnp.sqrt(jnp.sum(diffT * diffT, axis=0))


def _tc_post(acc2, deg0, deg1, zsem, W_pt, W_cls, b_gcn2, b_pt2, bcls2):
    return pl.pallas_call(
        _post_body,
        grid=(GRID,),
        in_specs=[
            pl.BlockSpec((BR, 2 * HID), lambda i: (i, 0)),
            pl.BlockSpec((BR,), lambda i: (i,)),
            pl.BlockSpec((BR,), lambda i: (i,)),
            pl.BlockSpec((ALIGN, BR), lambda i: (0, i)),
            pl.BlockSpec((HID, ALIGN), lambda i: (0, 0)),
            pl.BlockSpec((ALIGN, NUM_CLASSES), lambda i: (0, 0)),
            pl.BlockSpec((1, HID), lambda i: (0, 0)),
            pl.BlockSpec((1, ALIGN), lambda i: (0, 0)),
            pl.BlockSpec((1, NUM_CLASSES), lambda i: (0, 0)),
        ],
        out_specs=[
            pl.BlockSpec((ALIGN, BR), lambda i: (0, i)),
            pl.BlockSpec((NUM_CLASSES, BR), lambda i: (0, i)),
            pl.BlockSpec((BR,), lambda i: (i,)),
        ],
        out_shape=[
            jax.ShapeDtypeStruct((ALIGN, N), jnp.float32),
            jax.ShapeDtypeStruct((NUM_CLASSES, N), jnp.float32),
            jax.ShapeDtypeStruct((N,), jnp.float32),
        ],
    )(acc2, deg0, deg1, zsem, W_pt, W_cls, b_gcn2, b_pt2, bcls2)


def kernel(x, edge_index, W_gcn, b_gcn, W_pt, b_pt, W_ps, b_ps, W_cls, b_cls):
    f32 = jnp.float32
    src2d, dst2d = _tc_etr(edge_index)

    deg0, deg1 = _sc_degree(dst2d, jnp.zeros((DEGPAD,), f32))

    y, zsemT = _tc_pre(x, deg0, deg1, W_gcn, W_ps, b_ps.reshape(1, ALIGN))

    acc2 = _sc_scatter(y, src2d, dst2d, jnp.zeros((N, HID), f32))

    ztT, lgT, anomaly = _tc_post(
        acc2, deg0, deg1, zsemT, W_pt, W_cls,
        b_gcn.reshape(1, HID), b_pt.reshape(1, ALIGN),
        b_cls.reshape(1, NUM_CLASSES))

    return (lgT.T, anomaly, ztT.T, zsemT.T)


# speedup vs baseline: 2.2799x; 2.2799x over previous
import functools

import jax
import jax.numpy as jnp
from jax import lax
from jax.experimental import pallas as pl
from jax.experimental.pallas import tpu as pltpu
from jax.experimental.pallas import tpu_sc as plsc

N = 10000
E = 320000
IN_DIM = 128
HID = 64
ALIGN = 32
NUM_CLASSES = 7

NC = 2
NS = 16
NW = NC * NS

DEGPAD = 10240
DROWS = DEGPAD // NS
RPT = N // NS
CH = 128
NCHUNK = 2560
CPW = NCHUNK // NW
ACC_ROWS = DEGPAD
NB = 4
EBLK = 32768

BR = 2048
GRID = (N + BR - 1) // BR


def _sc_mesh():
    return plsc.VectorSubcoreMesh(core_axis_name="c", subcore_axis_name="s")


def _etr_body(e_ref, src_ref, dstd_ref, dsts_ref):
    i = pl.program_id(0)
    rows = EBLK // 128
    chunk2 = lax.broadcasted_iota(jnp.int32, (rows, 128), 0)
    lane = lax.broadcasted_iota(jnp.int32, (rows, 128), 1)
    elem = i * EBLK + chunk2 * 128 + lane
    valid = elem < E
    s2 = jnp.reshape(e_ref[0], (rows, 128))
    d2 = jnp.reshape(e_ref[1], (rows, 128))
    src_ref[...] = jnp.where(valid, s2, N + lane)
    dstd_ref[...] = jnp.where(valid, d2, N + lane)
    dsts_ref[...] = jnp.where(valid, d2, lax.rem(elem, N))


def _tc_etr(edge_index):
    rows = EBLK // 128
    return pl.pallas_call(
        _etr_body,
        grid=(NCHUNK // rows,),
        in_specs=[pl.BlockSpec((2, EBLK), lambda i: (0, i))],
        out_specs=[
            pl.BlockSpec((rows, 128), lambda i: (i, 0)),
            pl.BlockSpec((rows, 128), lambda i: (i, 0)),
            pl.BlockSpec((rows, 128), lambda i: (i, 0)),
        ],
        out_shape=[
            jax.ShapeDtypeStruct((NCHUNK, 128), jnp.int32),
            jax.ShapeDtypeStruct((NCHUNK, 128), jnp.int32),
            jax.ShapeDtypeStruct((NCHUNK, 128), jnp.int32),
        ],
    )(edge_index)


def _deg_body(dst_hbm, zeros_hbm, out0_hbm, out1_hbm, idx_v, ones_v, acc_sh, isem, asem):
    c = lax.axis_index("c")
    s = lax.axis_index("s")
    w = s * NC + c
    rslice = pl.ds(s * DROWS, DROWS)
    for i in range(8):
        ones_v[pl.ds(i * 16, 16)] = jnp.ones((16,), jnp.float32)
    pltpu.async_copy(zeros_hbm.at[rslice], acc_sh.at[rslice], isem).wait()
    pltpu.sync_copy(dst_hbm.at[pl.ds(w * CPW, CPW)], idx_v)
    plsc.subcore_barrier()

    @pl.loop(0, CPW, step=NB)
    def _chunks(t):
        hs = [
            pltpu.async_copy(ones_v, acc_sh.at[idx_v.at[t + b]],
                             asem, add=True)
            for b in range(NB)
        ]
        for h in hs:
            h.wait()

    plsc.subcore_barrier()

    @pl.when(c == 0)
    def _():
        pltpu.sync_copy(acc_sh.at[rslice], out0_hbm.at[rslice])

    @pl.when(c != 0)
    def _():
        pltpu.sync_copy(acc_sh.at[rslice], out1_hbm.at[rslice])


def _sc_degree(dst2d, zeros1d):
    return pl.kernel(
        _deg_body,
        out_type=[jax.ShapeDtypeStruct((DEGPAD,), jnp.float32),
                  jax.ShapeDtypeStruct((DEGPAD,), jnp.float32)],
        mesh=_sc_mesh(),
        compiler_params=pltpu.CompilerParams(use_tc_tiling_on_sc=False),
        scratch_types=[
            pltpu.VMEM((CPW, CH), jnp.int32),
            pltpu.VMEM((128,), jnp.float32),
            pltpu.VMEM_SHARED((DEGPAD,), jnp.float32),
            pltpu.SemaphoreType.DMA,
            pltpu.SemaphoreType.DMA,
        ],
    )(dst2d, zeros1d)


def _scat_body(y_hbm, src_hbm, dst_hbm, zeros_hbm, out_hbm,
               src_v, dst_v, rows_v, acc_sh, isem, gsem0, gsem1, ssem0, ssem1):
    c = lax.axis_index("c")
    s = lax.axis_index("s")
    w = s * NC + c
    rslice = pl.ds(s * RPT, RPT)
    gsems = (gsem0, gsem1)
    ssems = (ssem0, ssem1)
    NG = CPW // NB

    def fire_g(g, par):
        for b in range(NB):
            pltpu.async_copy(y_hbm.at[src_v.at[NB * g + b]],
                             rows_v.at[par * NB + b], gsems[par])

    def drain_g(g, par):
        for b in range(NB):
            pltpu.make_async_copy(y_hbm.at[src_v.at[NB * g + b]],
                                  rows_v.at[par * NB + b], gsems[par]).wait()

    def fire_s(g, par):
        for b in range(NB):
            pltpu.async_copy(rows_v.at[par * NB + b],
                             acc_sh.at[dst_v.at[NB * g + b]], ssems[par],
                             add=True)

    def drain_s(g, par):
        for b in range(NB):
            pltpu.make_async_copy(rows_v.at[par * NB + b],
                                  acc_sh.at[dst_v.at[NB * g + b]],
                                  ssems[par]).wait()

    arows = ACC_ROWS // NS
    ilo = s * arows

    @pl.when(c == 0)
    def _():
        pltpu.async_copy(y_hbm.at[pl.ds(ilo, arows)],
                         acc_sh.at[pl.ds(ilo, arows)], isem).wait()

    @pl.when(c != 0)
    def _():
        pltpu.async_copy(zeros_hbm.at[pl.ds(ilo, arows)],
                         acc_sh.at[pl.ds(ilo, arows)], isem).wait()

    pltpu.sync_copy(src_hbm.at[pl.ds(w * CPW, CPW)], src_v)
    pltpu.sync_copy(dst_hbm.at[pl.ds(w * CPW, CPW)], dst_v)
    plsc.subcore_barrier()

    fire_g(0, 0)
    fire_g(1, 1)
    drain_g(0, 0)
    fire_s(0, 0)

    @pl.loop(2, NG, step=2)
    def _groups(g):
        drain_s(g - 2, 0)
        fire_g(g, 0)
        drain_g(g - 1, 1)
        fire_s(g - 1, 1)
        drain_s(g - 1, 1)
        fire_g(g + 1, 1)
        drain_g(g, 0)
        fire_s(g, 0)

    drain_s(NG - 2, 0)
    drain_g(NG - 1, 1)
    fire_s(NG - 1, 1)
    drain_s(NG - 1, 1)

    plsc.subcore_barrier()
    pltpu.sync_copy(acc_sh.at[rslice],
                    out_hbm.at[pl.ds(s * RPT, RPT), pl.ds(c * HID, HID)])


def _sc_scatter(y, src2d, dst2d, zeros2d):
    return pl.kernel(
        _scat_body,
        out_type=jax.ShapeDtypeStruct((N, 2 * HID), jnp.float32),
        mesh=_sc_mesh(),
        compiler_params=pltpu.CompilerParams(use_tc_tiling_on_sc=False),
        scratch_types=[
            pltpu.VMEM((CPW, CH), jnp.int32),
            pltpu.VMEM((CPW, CH), jnp.int32),
            pltpu.VMEM((2 * NB, CH, HID), jnp.float32),
            pltpu.VMEM_SHARED((ACC_ROWS, HID), jnp.float32),
            pltpu.SemaphoreType.DMA,
            pltpu.SemaphoreType.DMA,
            pltpu.SemaphoreType.DMA,
            pltpu.SemaphoreType.DMA,
            pltpu.SemaphoreType.DMA,
        ],
    )(y, src2d, dst2d, zeros2d)


def _pre_body(x_ref, deg0_ref, deg1_ref, wg_ref, wps_ref, bps_ref,
              y_ref, zsem_ref):
    i = pl.program_id(0)
    deg = deg0_ref[...] + deg1_ref[...] + 1.0
    dinv = lax.rsqrt(deg)
    xb = x_ref[...]
    xw = jnp.dot(xb, wg_ref[...], preferred_element_type=jnp.float32)
    rowg = i * BR + lax.broadcasted_iota(jnp.int32, (BR, HID), 0)
    y_ref[...] = jnp.where(rowg < N, xw * dinv[:, None], 0.0)
    zs = (jnp.dot(xb, wps_ref[...], preferred_element_type=jnp.float32)
          + bps_ref[...])
    zsem_ref[...] = zs.T


def _tc_pre(xp, deg0, deg1, W_gcn, W_ps, b_ps2):
    return pl.pallas_call(
        _pre_body,
        grid=(GRID,),
        in_specs=[
            pl.BlockSpec((BR, IN_DIM), lambda i: (i, 0)),
            pl.BlockSpec((BR,), lambda i: (i,)),
            pl.BlockSpec((BR,), lambda i: (i,)),
            pl.BlockSpec((IN_DIM, HID), lambda i: (0, 0)),
            pl.BlockSpec((IN_DIM, ALIGN), lambda i: (0, 0)),
            pl.BlockSpec((1, ALIGN), lambda i: (0, 0)),
        ],
        out_specs=[
            pl.BlockSpec((BR, HID), lambda i: (i, 0)),
            pl.BlockSpec((ALIGN, BR), lambda i: (0, i)),
        ],
        out_shape=[
            jax.ShapeDtypeStruct((ACC_ROWS, HID), jnp.float32),
            jax.ShapeDtypeStruct((ALIGN, N), jnp.float32),
        ],
    )(xp, deg0, deg1, W_gcn, W_ps, b_ps2)


def _post_body(acc_ref, deg0_ref, deg1_ref, zsem_ref, wpt_ref, wcls_ref,
               bg_ref, bpt_ref, bcls_ref, zt_ref, lg_ref, an_ref):
    deg = deg0_ref[...] + deg1_ref[...] + 1.0
    dinv = lax.rsqrt(deg)
    a2 = acc_ref[...]
    agg = (a2[:, :HID] + a2[:, HID:]) * dinv[:, None] + bg_ref[...]
    h = jnp.maximum(agg, 0.0)
    zt = jnp.dot(h, wpt_ref[...], preferred_element_type=jnp.float32) + bpt_ref[...]
    ztT = zt.T
    zt_ref[...] = ztT
    lg = (jnp.dot(zt, wcls_ref[...], preferred_element_type=jnp.float32)
          + bcls_ref[...])
    lg_ref[...] = lg.T
    diffT = ztT - zsem_ref[...]
    an_ref[...] = jnp.sqrt(jnp.sum(diffT * diffT, axis=0))


def _tc_post(acc2, deg0, deg1, zsem, W_pt, W_cls, b_gcn2, b_pt2, bcls2):
    return pl.pallas_call(
        _post_body,
        grid=(GRID,),
        in_specs=[
            pl.BlockSpec((BR, 2 * HID), lambda i: (i, 0)),
            pl.BlockSpec((BR,), lambda i: (i,)),
            pl.BlockSpec((BR,), lambda i: (i,)),
            pl.BlockSpec((ALIGN, BR), lambda i: (0, i)),
            pl.BlockSpec((HID, ALIGN), lambda i: (0, 0)),
            pl.BlockSpec((ALIGN, NUM_CLASSES), lambda i: (0, 0)),
            pl.BlockSpec((1, HID), lambda i: (0, 0)),
            pl.BlockSpec((1, ALIGN), lambda i: (0, 0)),
            pl.BlockSpec((1, NUM_CLASSES), lambda i: (0, 0)),
        ],
        out_specs=[
            pl.BlockSpec((ALIGN, BR), lambda i: (0, i)),
            pl.BlockSpec((NUM_CLASSES, BR), lambda i: (0, i)),
            pl.BlockSpec((BR,), lambda i: (i,)),
        ],
        out_shape=[
            jax.ShapeDtypeStruct((ALIGN, N), jnp.float32),
            jax.ShapeDtypeStruct((NUM_CLASSES, N), jnp.float32),
            jax.ShapeDtypeStruct((N,), jnp.float32),
        ],
    )(acc2, deg0, deg1, zsem, W_pt, W_cls, b_gcn2, b_pt2, bcls2)


def kernel(x, edge_index, W_gcn, b_gcn, W_pt, b_pt, W_ps, b_ps, W_cls, b_cls):
    f32 = jnp.float32
    src2d, dstd, dsts = _tc_etr(edge_index)

    deg0, deg1 = _sc_degree(dstd, jnp.zeros((DEGPAD,), f32))

    y, zsemT = _tc_pre(x, deg0, deg1, W_gcn, W_ps, b_ps.reshape(1, ALIGN))

    acc2 = _sc_scatter(y, src2d, dsts, jnp.zeros((ACC_ROWS, HID), f32))

    ztT, lgT, anomaly = _tc_post(
        acc2, deg0, deg1, zsemT, W_pt, W_cls,
        b_gcn.reshape(1, HID), b_pt.reshape(1, ALIGN),
        b_cls.reshape(1, NUM_CLASSES))

    return (lgT.T, anomaly, ztT.T, zsemT.T)
